# baseline (speedup 1.0000x reference)
import functools

import jax
import jax.numpy as jnp
from jax import lax
from jax.experimental import pallas as pl
from jax.experimental.pallas import tpu as pltpu
from jax.experimental.pallas import tpu_sc as plsc

_TILE = 256


def _router_alpha(x2d, router_w, shg_w, shg_b2):
    N, D = x2d.shape
    E = router_w.shape[0]
    TB = 1024

    def body(x_ref, w_ref, sg_ref, sb_ref, eo_ref, ao_ref):
        xf = x_ref[...]
        logits = lax.dot_general(
            xf.astype(jnp.bfloat16), w_ref[...].astype(jnp.bfloat16),
            (((1,), (1,)), ((), ())),
            preferred_element_type=jnp.float32)
        maxv = jnp.max(logits, axis=1, keepdims=True)
        ids = lax.broadcasted_iota(jnp.int32, logits.shape, 1)
        eo_ref[...] = jnp.min(jnp.where(logits >= maxv, ids, E),
                              axis=1, keepdims=True)
        glogit = jnp.sum(xf * sg_ref[...], axis=1, keepdims=True)
        ao_ref[...] = jax.nn.sigmoid(glogit + sb_ref[0, 0])

    eo, ao = pl.pallas_call(
        body,
        grid=(N // TB,),
        in_specs=[pl.BlockSpec((TB, D), lambda i: (i, 0)),
                  pl.BlockSpec((E, D), lambda i: (0, 0)),
                  pl.BlockSpec((1, D), lambda i: (0, 0)),
                  pl.BlockSpec((1, 1), lambda i: (0, 0))],
        out_specs=[pl.BlockSpec((TB, 1), lambda i: (i, 0)),
                   pl.BlockSpec((TB, 1), lambda i: (i, 0))],
        out_shape=[jax.ShapeDtypeStruct((N, 1), jnp.int32),
                   jax.ShapeDtypeStruct((N, 1), jnp.float32)],
    )(x2d, router_w, shg_w, shg_b2)
    return eo[:, 0], ao


def _build_dispatch(e_idx, E, T, NPAD):
    N = e_idx.shape[0]
    onehot = (e_idx[:, None] == jnp.arange(E, dtype=e_idx.dtype)[None, :]
              ).astype(jnp.int32)
    ranks = jnp.cumsum(onehot, axis=0) - 1
    rank_t = jnp.take_along_axis(ranks, e_idx[:, None], axis=1)[:, 0]
    counts = jnp.sum(onehot, axis=0)
    padded = ((counts + T - 1) // T) * T
    p_end = jnp.cumsum(padded)
    p_off = p_end - padded
    dst = (p_off[e_idx] + rank_t).astype(jnp.int32)
    src = jnp.zeros((NPAD,), jnp.int32).at[dst].set(
        jnp.arange(N, dtype=jnp.int32))
    nt = NPAD // T
    tile_expert = jnp.searchsorted(
        p_end, jnp.arange(nt, dtype=p_end.dtype) * T, side='right')
    tile_expert = jnp.minimum(tile_expert, E - 1).astype(jnp.int32)
    return dst, src, tile_expert


def _sc_gather(table, idx):
    V, D = table.shape
    Bn = idx.shape[0]
    info = plsc.get_sparse_core_info()
    NC = info.num_cores
    NW = NC * info.num_subcores
    bpw = Bn // NW
    CH = 32
    NB = 3
    nch = bpw // CH
    AHEAD = NB - 1
    mesh = plsc.VectorSubcoreMesh(core_axis_name="c", subcore_axis_name="s")

    @functools.partial(
        pl.kernel, mesh=mesh,
        out_type=jax.ShapeDtypeStruct((Bn, D), table.dtype),
        scratch_types=[pltpu.VMEM((bpw,), jnp.int32),
                       pltpu.VMEM((NB, CH, D), table.dtype)]
        + [pltpu.SemaphoreType.DMA] * (2 * NB))
    def gk(table_hbm, idx_hbm, out_hbm, idx_v, bufs, *sems):
        gsem = sems[:NB]
        wsem = sems[NB:]
        wid = lax.axis_index("s") * NC + lax.axis_index("c")
        base = wid * bpw
        pltpu.sync_copy(idx_hbm.at[pl.ds(base, bpw)], idx_v)
        gds = [None] * nch
        wds = [None] * nch
        for c in range(nch):
            b = c % NB
            if c >= NB:
                wds[c - NB].wait()
            gds[c] = pltpu.async_copy(
                table_hbm.at[idx_v.at[pl.ds(c * CH, CH)]],
                bufs.at[b], gsem[b])
            p = c - AHEAD
            if p >= 0:
                gds[p].wait()
                wds[p] = pltpu.async_copy(
                    bufs.at[p % NB], out_hbm.at[pl.ds(base + p * CH, CH)],
                    wsem[p % NB])
        for p in range(max(0, nch - AHEAD), nch):
            gds[p].wait()
            wds[p] = pltpu.async_copy(
                bufs.at[p % NB], out_hbm.at[pl.ds(base + p * CH, CH)],
                wsem[p % NB])
        for p in range(max(0, nch - NB), nch):
            wds[p].wait()

    return gk(table, idx)


def _grouped_swiglu(tile_expert, x_sorted, egate_bf, eup_bf, edown_bf):
    NPAD, D = x_sorted.shape
    E, F, _ = egate_bf.shape
    nt = NPAD // _TILE

    def body(te_ref, xs_ref, gw_ref, uw_ref, dw_ref, o_ref):
        xb = xs_ref[...].astype(jnp.bfloat16)
        g = lax.dot_general(xb, gw_ref[0], (((1,), (1,)), ((), ())),
                            preferred_element_type=jnp.float32)
        u = lax.dot_general(xb, uw_ref[0], (((1,), (1,)), ((), ())),
                            preferred_element_type=jnp.float32)
        h = (g * jax.nn.sigmoid(g) * u).astype(jnp.bfloat16)
        o_ref[...] = lax.dot_general(h, dw_ref[0], (((1,), (1,)), ((), ())),
                                     preferred_element_type=jnp.float32)

    grid_spec = pltpu.PrefetchScalarGridSpec(
        num_scalar_prefetch=1,
        grid=(nt,),
        in_specs=[pl.BlockSpec((_TILE, D), lambda i, te: (i, 0)),
                  pl.BlockSpec((1, F, D), lambda i, te: (te[i], 0, 0)),
                  pl.BlockSpec((1, F, D), lambda i, te: (te[i], 0, 0)),
                  pl.BlockSpec((1, D, F), lambda i, te: (te[i], 0, 0))],
        out_specs=pl.BlockSpec((_TILE, D), lambda i, te: (i, 0)),
    )
    return pl.pallas_call(
        body, grid_spec=grid_spec,
        out_shape=jax.ShapeDtypeStruct((NPAD, D), jnp.float32),
    )(tile_expert, x_sorted, egate_bf, eup_bf, edown_bf)


def _shared_swiglu(x2d, gw_bf, uw_bf, dw_bf):
    N, D = x2d.shape
    F = gw_bf.shape[0]
    TB = 256

    def body(x_ref, gw_ref, uw_ref, dw_ref, y_ref):
        xb = x_ref[...].astype(jnp.bfloat16)
        g = lax.dot_general(xb, gw_ref[...], (((1,), (1,)), ((), ())),
                            preferred_element_type=jnp.float32)
        u = lax.dot_general(xb, uw_ref[...], (((1,), (1,)), ((), ())),
                            preferred_element_type=jnp.float32)
        h = (g * jax.nn.sigmoid(g) * u).astype(jnp.bfloat16)
        y_ref[...] = lax.dot_general(h, dw_ref[...], (((1,), (1,)), ((), ())),
                                     preferred_element_type=jnp.float32)

    return pl.pallas_call(
        body,
        grid=(N // TB,),
        in_specs=[pl.BlockSpec((TB, D), lambda i: (i, 0)),
                  pl.BlockSpec((F, D), lambda i: (0, 0)),
                  pl.BlockSpec((F, D), lambda i: (0, 0)),
                  pl.BlockSpec((D, F), lambda i: (0, 0))],
        out_specs=pl.BlockSpec((TB, D), lambda i: (i, 0)),
        out_shape=jax.ShapeDtypeStruct((N, D), jnp.float32),
    )(x2d, gw_bf, uw_bf, dw_bf)


def _combine(alpha, sh_out, routed):
    N, D = sh_out.shape
    TB = 512

    def body(a_ref, s_ref, r_ref, y_ref):
        a = a_ref[...]
        y_ref[...] = a * s_ref[...] + (1.0 - a) * r_ref[...]

    return pl.pallas_call(
        body,
        grid=(N // TB,),
        in_specs=[pl.BlockSpec((TB, 1), lambda i: (i, 0)),
                  pl.BlockSpec((TB, D), lambda i: (i, 0)),
                  pl.BlockSpec((TB, D), lambda i: (i, 0))],
        out_specs=pl.BlockSpec((TB, D), lambda i: (i, 0)),
        out_shape=jax.ShapeDtypeStruct((N, D), jnp.float32),
    )(alpha, sh_out, routed)


def kernel(x, router_w, egate_w, eup_w, edown_w,
           sh_gate_w, sh_up_w, sh_down_w, shg_w, shg_b):
    B, S, D = x.shape
    N = B * S
    E = router_w.shape[0]
    x2d = x.reshape(N, D)
    NPAD = N + E * _TILE

    e_idx, alpha = _router_alpha(x2d, router_w, shg_w, shg_b.reshape(1, 1))
    dst, src, tile_expert = _build_dispatch(e_idx, E, _TILE, NPAD)

    x_sorted = _sc_gather(x2d, src)
    out_pad = _grouped_swiglu(tile_expert, x_sorted,
                              egate_w.astype(jnp.bfloat16),
                              eup_w.astype(jnp.bfloat16),
                              edown_w.astype(jnp.bfloat16))
    routed = _sc_gather(out_pad, dst)

    sh_out = _shared_swiglu(x2d,
                            sh_gate_w.astype(jnp.bfloat16),
                            sh_up_w.astype(jnp.bfloat16),
                            sh_down_w.astype(jnp.bfloat16))
    y2d = _combine(alpha, sh_out, routed)
    return y2d.reshape(B, S, D)


# speedup vs baseline: 1.3528x; 1.3528x over previous
import functools

import jax
import jax.numpy as jnp
from jax import lax
from jax.experimental import pallas as pl
from jax.experimental.pallas import tpu as pltpu
from jax.experimental.pallas import tpu_sc as plsc

_TILE = 256


def _router_alpha(x2d, router_w, shg_w, shg_b2):
    N, D = x2d.shape
    E = router_w.shape[0]
    TB = 1024

    def body(x_ref, w_ref, sg_ref, sb_ref, eo_ref, ao_ref):
        xf = x_ref[...]
        logits = lax.dot_general(
            xf.astype(jnp.bfloat16), w_ref[...].astype(jnp.bfloat16),
            (((1,), (1,)), ((), ())),
            preferred_element_type=jnp.float32)
        maxv = jnp.max(logits, axis=1, keepdims=True)
        ids = lax.broadcasted_iota(jnp.int32, logits.shape, 1)
        eo_ref[...] = jnp.min(jnp.where(logits >= maxv, ids, E),
                              axis=1, keepdims=True)
        glogit = jnp.sum(xf * sg_ref[...], axis=1, keepdims=True)
        ao_ref[...] = jax.nn.sigmoid(glogit + sb_ref[0, 0])

    eo, ao = pl.pallas_call(
        body,
        grid=(N // TB,),
        in_specs=[pl.BlockSpec((TB, D), lambda i: (i, 0)),
                  pl.BlockSpec((E, D), lambda i: (0, 0)),
                  pl.BlockSpec((1, D), lambda i: (0, 0)),
                  pl.BlockSpec((1, 1), lambda i: (0, 0))],
        out_specs=[pl.BlockSpec((TB, 1), lambda i: (i, 0)),
                   pl.BlockSpec((TB, 1), lambda i: (i, 0))],
        out_shape=[jax.ShapeDtypeStruct((N, 1), jnp.int32),
                   jax.ShapeDtypeStruct((N, 1), jnp.float32)],
    )(x2d, router_w, shg_w, shg_b2)
    return eo[:, 0], ao


def _build_dispatch(e_idx, E, T, NPAD):
    N = e_idx.shape[0]
    onehot = (e_idx[:, None] == jnp.arange(E, dtype=e_idx.dtype)[None, :]
              ).astype(jnp.int32)
    ranks = jnp.cumsum(onehot, axis=0) - 1
    rank_t = jnp.take_along_axis(ranks, e_idx[:, None], axis=1)[:, 0]
    counts = jnp.sum(onehot, axis=0)
    padded = ((counts + T - 1) // T) * T
    p_end = jnp.cumsum(padded)
    p_off = p_end - padded
    dst = (p_off[e_idx] + rank_t).astype(jnp.int32)
    src = (jnp.arange(NPAD, dtype=jnp.int32) % N).at[dst].set(
        jnp.arange(N, dtype=jnp.int32))
    nt = NPAD // T
    tile_expert = jnp.searchsorted(
        p_end, jnp.arange(nt, dtype=p_end.dtype) * T, side='right')
    tile_expert = jnp.minimum(tile_expert, E - 1).astype(jnp.int32)
    return dst, src, tile_expert


def _sc_gather(table, idx):
    V, D = table.shape
    Bn = idx.shape[0]
    info = plsc.get_sparse_core_info()
    NC = info.num_cores
    NW = NC * info.num_subcores
    bpw = Bn // NW
    CH = 32
    NB = 3
    nch = bpw // CH
    AHEAD = NB - 1
    mesh = plsc.VectorSubcoreMesh(core_axis_name="c", subcore_axis_name="s")

    @functools.partial(
        pl.kernel, mesh=mesh,
        out_type=jax.ShapeDtypeStruct((Bn, D), table.dtype),
        scratch_types=[pltpu.VMEM((bpw,), jnp.int32),
                       pltpu.VMEM((NB, CH, D), table.dtype)]
        + [pltpu.SemaphoreType.DMA] * (2 * NB))
    def gk(table_hbm, idx_hbm, out_hbm, idx_v, bufs, *sems):
        gsem = sems[:NB]
        wsem = sems[NB:]
        wid = lax.axis_index("s") * NC + lax.axis_index("c")
        base = wid * bpw
        pltpu.sync_copy(idx_hbm.at[pl.ds(base, bpw)], idx_v)
        gds = [None] * nch
        wds = [None] * nch
        for c in range(nch):
            b = c % NB
            if c >= NB:
                wds[c - NB].wait()
            gds[c] = pltpu.async_copy(
                table_hbm.at[idx_v.at[pl.ds(c * CH, CH)]],
                bufs.at[b], gsem[b])
            p = c - AHEAD
            if p >= 0:
                gds[p].wait()
                wds[p] = pltpu.async_copy(
                    bufs.at[p % NB], out_hbm.at[pl.ds(base + p * CH, CH)],
                    wsem[p % NB])
        for p in range(max(0, nch - AHEAD), nch):
            gds[p].wait()
            wds[p] = pltpu.async_copy(
                bufs.at[p % NB], out_hbm.at[pl.ds(base + p * CH, CH)],
                wsem[p % NB])
        for p in range(max(0, nch - NB), nch):
            wds[p].wait()

    return gk(table, idx)


def _grouped_swiglu(tile_expert, x_sorted, egate_bf, eup_bf, edown_bf):
    NPAD, D = x_sorted.shape
    E, F, _ = egate_bf.shape
    nt = NPAD // _TILE

    def body(te_ref, xs_ref, gw_ref, uw_ref, dw_ref, o_ref):
        xb = xs_ref[...].astype(jnp.bfloat16)
        g = lax.dot_general(xb, gw_ref[0], (((1,), (1,)), ((), ())),
                            preferred_element_type=jnp.float32)
        u = lax.dot_general(xb, uw_ref[0], (((1,), (1,)), ((), ())),
                            preferred_element_type=jnp.float32)
        h = (g * jax.nn.sigmoid(g) * u).astype(jnp.bfloat16)
        o_ref[...] = lax.dot_general(h, dw_ref[0], (((1,), (1,)), ((), ())),
                                     preferred_element_type=jnp.float32)

    grid_spec = pltpu.PrefetchScalarGridSpec(
        num_scalar_prefetch=1,
        grid=(nt,),
        in_specs=[pl.BlockSpec((_TILE, D), lambda i, te: (i, 0)),
                  pl.BlockSpec((1, F, D), lambda i, te: (te[i], 0, 0)),
                  pl.BlockSpec((1, F, D), lambda i, te: (te[i], 0, 0)),
                  pl.BlockSpec((1, D, F), lambda i, te: (te[i], 0, 0))],
        out_specs=pl.BlockSpec((_TILE, D), lambda i, te: (i, 0)),
    )
    return pl.pallas_call(
        body, grid_spec=grid_spec,
        out_shape=jax.ShapeDtypeStruct((NPAD, D), jnp.float32),
    )(tile_expert, x_sorted, egate_bf, eup_bf, edown_bf)


def _shared_combine(x2d, gw_bf, uw_bf, dw_bf, alpha, routed):
    N, D = x2d.shape
    F = gw_bf.shape[0]
    TB = 256

    def body(x_ref, gw_ref, uw_ref, dw_ref, a_ref, r_ref, y_ref):
        xb = x_ref[...].astype(jnp.bfloat16)
        g = lax.dot_general(xb, gw_ref[...], (((1,), (1,)), ((), ())),
                            preferred_element_type=jnp.float32)
        u = lax.dot_general(xb, uw_ref[...], (((1,), (1,)), ((), ())),
                            preferred_element_type=jnp.float32)
        h = (g * jax.nn.sigmoid(g) * u).astype(jnp.bfloat16)
        sh = lax.dot_general(h, dw_ref[...], (((1,), (1,)), ((), ())),
                             preferred_element_type=jnp.float32)
        a = a_ref[...]
        y_ref[...] = a * sh + (1.0 - a) * r_ref[...]

    return pl.pallas_call(
        body,
        grid=(N // TB,),
        in_specs=[pl.BlockSpec((TB, D), lambda i: (i, 0)),
                  pl.BlockSpec((F, D), lambda i: (0, 0)),
                  pl.BlockSpec((F, D), lambda i: (0, 0)),
                  pl.BlockSpec((D, F), lambda i: (0, 0)),
                  pl.BlockSpec((TB, 1), lambda i: (i, 0)),
                  pl.BlockSpec((TB, D), lambda i: (i, 0))],
        out_specs=pl.BlockSpec((TB, D), lambda i: (i, 0)),
        out_shape=jax.ShapeDtypeStruct((N, D), jnp.float32),
    )(x2d, gw_bf, uw_bf, dw_bf, alpha, routed)


def kernel(x, router_w, egate_w, eup_w, edown_w,
           sh_gate_w, sh_up_w, sh_down_w, shg_w, shg_b):
    B, S, D = x.shape
    N = B * S
    E = router_w.shape[0]
    x2d = x.reshape(N, D)
    NPAD = N + E * _TILE

    e_idx, alpha = _router_alpha(x2d, router_w, shg_w, shg_b.reshape(1, 1))
    dst, src, tile_expert = _build_dispatch(e_idx, E, _TILE, NPAD)

    x_sorted = _sc_gather(x2d, src)
    out_pad = _grouped_swiglu(tile_expert, x_sorted,
                              egate_w.astype(jnp.bfloat16),
                              eup_w.astype(jnp.bfloat16),
                              edown_w.astype(jnp.bfloat16))
    routed = _sc_gather(out_pad, dst)

    y2d = _shared_combine(x2d,
                          sh_gate_w.astype(jnp.bfloat16),
                          sh_up_w.astype(jnp.bfloat16),
                          sh_down_w.astype(jnp.bfloat16),
                          alpha, routed)
    return y2d.reshape(B, S, D)


# speedup vs baseline: 1.4833x; 1.0965x over previous
import functools

import jax
import jax.numpy as jnp
from jax import lax
from jax.experimental import pallas as pl
from jax.experimental.pallas import tpu as pltpu
from jax.experimental.pallas import tpu_sc as plsc

_TILE = 256


def _router_alpha(x2d, router_w, shg_w, shg_b2):
    N, D = x2d.shape
    E = router_w.shape[0]
    TB = 1024

    def body(x_ref, w_ref, sg_ref, sb_ref, eo_ref, ao_ref):
        xf = x_ref[...]
        xb = xf.astype(jnp.bfloat16)
        logits = lax.dot_general(
            xb, w_ref[...].astype(jnp.bfloat16),
            (((1,), (1,)), ((), ())),
            preferred_element_type=jnp.float32)
        maxv = jnp.max(logits, axis=1, keepdims=True)
        ids = lax.broadcasted_iota(jnp.int32, logits.shape, 1)
        eo_ref[...] = jnp.min(jnp.where(logits >= maxv, ids, E),
                              axis=1, keepdims=True)
        glogit = jnp.sum(xf * sg_ref[...], axis=1, keepdims=True)
        ao_ref[...] = jax.nn.sigmoid(glogit + sb_ref[0, 0])

    eo, ao = pl.pallas_call(
        body,
        grid=(N // TB,),
        in_specs=[pl.BlockSpec((TB, D), lambda i: (i, 0)),
                  pl.BlockSpec((E, D), lambda i: (0, 0)),
                  pl.BlockSpec((1, D), lambda i: (0, 0)),
                  pl.BlockSpec((1, 1), lambda i: (0, 0))],
        out_specs=[pl.BlockSpec((TB, 1), lambda i: (i, 0)),
                   pl.BlockSpec((TB, 1), lambda i: (i, 0))],
        out_shape=[jax.ShapeDtypeStruct((N, 1), jnp.int32),
                   jax.ShapeDtypeStruct((N, 1), jnp.float32)],
    )(x2d, router_w, shg_w, shg_b2)
    return eo[:, 0], ao


def _build_dispatch(e_idx, E, T, NPAD):
    N = e_idx.shape[0]
    onehot = (e_idx[:, None] == jnp.arange(E, dtype=e_idx.dtype)[None, :]
              ).astype(jnp.int32)
    ranks = jnp.cumsum(onehot, axis=0) - 1
    rank_t = jnp.take_along_axis(ranks, e_idx[:, None], axis=1)[:, 0]
    counts = jnp.sum(onehot, axis=0)
    padded = ((counts + T - 1) // T) * T
    p_end = jnp.cumsum(padded)
    p_off = p_end - padded
    dst = (p_off[e_idx] + rank_t).astype(jnp.int32)
    src = (jnp.arange(NPAD, dtype=jnp.int32) % N).at[dst].set(
        jnp.arange(N, dtype=jnp.int32))
    nt = NPAD // T
    tile_expert = jnp.searchsorted(
        p_end, jnp.arange(nt, dtype=p_end.dtype) * T, side='right')
    tile_expert = jnp.minimum(tile_expert, E - 1).astype(jnp.int32)
    return dst, src, tile_expert


def _sc_gather(table, idx):
    V, D = table.shape
    Bn = idx.shape[0]
    info = plsc.get_sparse_core_info()
    NC = info.num_cores
    NW = NC * info.num_subcores
    bpw = Bn // NW
    CH = 64 if table.dtype == jnp.bfloat16 else 32
    NB = 3
    nch = bpw // CH
    AHEAD = NB - 1
    mesh = plsc.VectorSubcoreMesh(core_axis_name="c", subcore_axis_name="s")

    @functools.partial(
        pl.kernel, mesh=mesh,
        out_type=jax.ShapeDtypeStruct((Bn, D), table.dtype),
        scratch_types=[pltpu.VMEM((bpw,), jnp.int32),
                       pltpu.VMEM((NB, CH, D), table.dtype)]
        + [pltpu.SemaphoreType.DMA] * (2 * NB))
    def gk(table_hbm, idx_hbm, out_hbm, idx_v, bufs, *sems):
        gsem = sems[:NB]
        wsem = sems[NB:]
        wid = lax.axis_index("s") * NC + lax.axis_index("c")
        base = wid * bpw
        pltpu.sync_copy(idx_hbm.at[pl.ds(base, bpw)], idx_v)
        gds = [None] * nch
        wds = [None] * nch
        for c in range(nch):
            b = c % NB
            if c >= NB:
                wds[c - NB].wait()
            gds[c] = pltpu.async_copy(
                table_hbm.at[idx_v.at[pl.ds(c * CH, CH)]],
                bufs.at[b], gsem[b])
            p = c - AHEAD
            if p >= 0:
                gds[p].wait()
                wds[p] = pltpu.async_copy(
                    bufs.at[p % NB], out_hbm.at[pl.ds(base + p * CH, CH)],
                    wsem[p % NB])
        for p in range(max(0, nch - AHEAD), nch):
            gds[p].wait()
            wds[p] = pltpu.async_copy(
                bufs.at[p % NB], out_hbm.at[pl.ds(base + p * CH, CH)],
                wsem[p % NB])
        for p in range(max(0, nch - NB), nch):
            wds[p].wait()

    return gk(table, idx)


def _grouped_swiglu(tile_expert, x_sorted, egate_w, eup_w, edown_w):
    NPAD, D = x_sorted.shape
    E, F, _ = egate_w.shape
    nt = NPAD // _TILE

    def body(te_ref, xs_ref, gw_ref, uw_ref, dw_ref, o_ref):
        xb = xs_ref[...].astype(jnp.bfloat16)
        g = lax.dot_general(xb, gw_ref[0].astype(jnp.bfloat16),
                            (((1,), (1,)), ((), ())),
                            preferred_element_type=jnp.float32)
        u = lax.dot_general(xb, uw_ref[0].astype(jnp.bfloat16),
                            (((1,), (1,)), ((), ())),
                            preferred_element_type=jnp.float32)
        h = (g * jax.nn.sigmoid(g) * u).astype(jnp.bfloat16)
        o_ref[...] = lax.dot_general(h, dw_ref[0].astype(jnp.bfloat16),
                                     (((1,), (1,)), ((), ())),
                                     preferred_element_type=jnp.float32)

    grid_spec = pltpu.PrefetchScalarGridSpec(
        num_scalar_prefetch=1,
        grid=(nt,),
        in_specs=[pl.BlockSpec((_TILE, D), lambda i, te: (i, 0)),
                  pl.BlockSpec((1, F, D), lambda i, te: (te[i], 0, 0)),
                  pl.BlockSpec((1, F, D), lambda i, te: (te[i], 0, 0)),
                  pl.BlockSpec((1, D, F), lambda i, te: (te[i], 0, 0))],
        out_specs=pl.BlockSpec((_TILE, D), lambda i, te: (i, 0)),
    )
    return pl.pallas_call(
        body, grid_spec=grid_spec,
        out_shape=jax.ShapeDtypeStruct((NPAD, D), jnp.float32),
    )(tile_expert, x_sorted, egate_w, eup_w, edown_w)


def _shared_combine(x2d, gw_bf, uw_bf, dw_bf, alpha, routed):
    N, D = x2d.shape
    F = gw_bf.shape[0]
    TB = 256

    def body(x_ref, gw_ref, uw_ref, dw_ref, a_ref, r_ref, y_ref):
        xb = x_ref[...].astype(jnp.bfloat16)
        g = lax.dot_general(xb, gw_ref[...], (((1,), (1,)), ((), ())),
                            preferred_element_type=jnp.float32)
        u = lax.dot_general(xb, uw_ref[...], (((1,), (1,)), ((), ())),
                            preferred_element_type=jnp.float32)
        h = (g * jax.nn.sigmoid(g) * u).astype(jnp.bfloat16)
        sh = lax.dot_general(h, dw_ref[...], (((1,), (1,)), ((), ())),
                             preferred_element_type=jnp.float32)
        a = a_ref[...]
        y_ref[...] = a * sh + (1.0 - a) * r_ref[...].astype(jnp.float32)

    return pl.pallas_call(
        body,
        grid=(N // TB,),
        in_specs=[pl.BlockSpec((TB, D), lambda i: (i, 0)),
                  pl.BlockSpec((F, D), lambda i: (0, 0)),
                  pl.BlockSpec((F, D), lambda i: (0, 0)),
                  pl.BlockSpec((D, F), lambda i: (0, 0)),
                  pl.BlockSpec((TB, 1), lambda i: (i, 0)),
                  pl.BlockSpec((TB, D), lambda i: (i, 0))],
        out_specs=pl.BlockSpec((TB, D), lambda i: (i, 0)),
        out_shape=jax.ShapeDtypeStruct((N, D), jnp.float32),
    )(x2d, gw_bf, uw_bf, dw_bf, alpha, routed)


def kernel(x, router_w, egate_w, eup_w, edown_w,
           sh_gate_w, sh_up_w, sh_down_w, shg_w, shg_b):
    B, S, D = x.shape
    N = B * S
    E = router_w.shape[0]
    x2d = x.reshape(N, D)
    NPAD = N + E * _TILE

    e_idx, alpha = _router_alpha(x2d, router_w, shg_w, shg_b.reshape(1, 1))
    dst, src, tile_expert = _build_dispatch(e_idx, E, _TILE, NPAD)

    x_sorted = _sc_gather(x2d, src)
    out_pad = _grouped_swiglu(tile_expert, x_sorted, egate_w, eup_w, edown_w)
    routed = _sc_gather(out_pad, dst)

    y2d = _shared_combine(x2d,
                          sh_gate_w.astype(jnp.bfloat16),
                          sh_up_w.astype(jnp.bfloat16),
                          sh_down_w.astype(jnp.bfloat16),
                          alpha, routed)
    return y2d.reshape(B, S, D)


# speedup vs baseline: 1.6076x; 1.0838x over previous
import functools

import jax
import jax.numpy as jnp
from jax import lax
from jax.experimental import pallas as pl
from jax.experimental.pallas import tpu as pltpu
from jax.experimental.pallas import tpu_sc as plsc

_TILE = 256


def _router_alpha(x2d, router_w, shg_w, shg_b2):
    N, D = x2d.shape
    E = router_w.shape[0]
    TB = 1024

    def body(x_ref, w_ref, sg_ref, sb_ref, eo_ref, ao_ref):
        xf = x_ref[...]
        xb = xf.astype(jnp.bfloat16)
        logits = lax.dot_general(
            xb, w_ref[...].astype(jnp.bfloat16),
            (((1,), (1,)), ((), ())),
            preferred_element_type=jnp.float32)
        maxv = jnp.max(logits, axis=1, keepdims=True)
        ids = lax.broadcasted_iota(jnp.int32, logits.shape, 1)
        eo_ref[...] = jnp.min(jnp.where(logits >= maxv, ids, E),
                              axis=1, keepdims=True)
        glogit = jnp.sum(xf * sg_ref[...], axis=1, keepdims=True)
        ao_ref[...] = jax.nn.sigmoid(glogit + sb_ref[0, 0])

    eo, ao = pl.pallas_call(
        body,
        grid=(N // TB,),
        in_specs=[pl.BlockSpec((TB, D), lambda i: (i, 0)),
                  pl.BlockSpec((E, D), lambda i: (0, 0)),
                  pl.BlockSpec((1, D), lambda i: (0, 0)),
                  pl.BlockSpec((1, 1), lambda i: (0, 0))],
        out_specs=[pl.BlockSpec((TB, 1), lambda i: (i, 0)),
                   pl.BlockSpec((TB, 1), lambda i: (i, 0))],
        out_shape=[jax.ShapeDtypeStruct((N, 1), jnp.int32),
                   jax.ShapeDtypeStruct((N, 1), jnp.float32)],
    )(x2d, router_w, shg_w, shg_b2)
    return eo[:, 0], ao


def _build_dispatch(e_idx, E, T, NPAD):
    N = e_idx.shape[0]
    onehot = (e_idx[:, None] == jnp.arange(E, dtype=e_idx.dtype)[None, :]
              ).astype(jnp.int32)
    ranks = jnp.cumsum(onehot, axis=0) - 1
    rank_t = jnp.take_along_axis(ranks, e_idx[:, None], axis=1)[:, 0]
    counts = jnp.sum(onehot, axis=0)
    padded = ((counts + T - 1) // T) * T
    p_end = jnp.cumsum(padded)
    p_off = p_end - padded
    dst = (p_off[e_idx] + rank_t).astype(jnp.int32)
    nt = NPAD // T
    tile_expert = jnp.searchsorted(
        p_end, jnp.arange(nt, dtype=p_end.dtype) * T, side='right')
    tile_expert = jnp.minimum(tile_expert, E - 1).astype(jnp.int32)
    return dst, tile_expert


def _sc_scatter_rows(x2d, dst3d, NPAD):
    N, D = x2d.shape
    NW, nch, CH = dst3d.shape
    info = plsc.get_sparse_core_info()
    NC = info.num_cores
    NB = 3
    AHEAD = NB - 1
    bpw = nch * CH
    mesh = plsc.VectorSubcoreMesh(core_axis_name="c", subcore_axis_name="s")

    @functools.partial(
        pl.kernel, mesh=mesh,
        out_type=jax.ShapeDtypeStruct((NPAD, D), x2d.dtype),
        scratch_types=[pltpu.VMEM((nch, CH), jnp.int32),
                       pltpu.VMEM((NB, CH, D), x2d.dtype)]
        + [pltpu.SemaphoreType.DMA] * (2 * NB))
    def sk(x_hbm, idx_hbm, out_hbm, idx_v, bufs, *sems):
        gsem = sems[:NB]
        wsem = sems[NB:]
        wid = lax.axis_index("s") * NC + lax.axis_index("c")
        base = wid * bpw
        pltpu.sync_copy(idx_hbm.at[wid], idx_v)
        gds = [None] * nch
        wds = [None] * nch
        for c in range(nch):
            b = c % NB
            if c >= NB:
                wds[c - NB].wait()
            gds[c] = pltpu.async_copy(
                x_hbm.at[pl.ds(base + c * CH, CH)], bufs.at[b], gsem[b])
            p = c - AHEAD
            if p >= 0:
                gds[p].wait()
                wds[p] = pltpu.async_copy(
                    bufs.at[p % NB], out_hbm.at[idx_v.at[p]], wsem[p % NB])
        for p in range(max(0, nch - AHEAD), nch):
            gds[p].wait()
            wds[p] = pltpu.async_copy(
                bufs.at[p % NB], out_hbm.at[idx_v.at[p]], wsem[p % NB])
        for p in range(max(0, nch - NB), nch):
            wds[p].wait()

    return sk(x2d, dst3d)


def _sc_gather(table, idx):
    V, D = table.shape
    Bn = idx.shape[0]
    info = plsc.get_sparse_core_info()
    NC = info.num_cores
    NW = NC * info.num_subcores
    bpw = Bn // NW
    CH = 64 if table.dtype == jnp.bfloat16 else 32
    NB = 3
    nch = bpw // CH
    AHEAD = NB - 1
    mesh = plsc.VectorSubcoreMesh(core_axis_name="c", subcore_axis_name="s")

    @functools.partial(
        pl.kernel, mesh=mesh,
        out_type=jax.ShapeDtypeStruct((Bn, D), table.dtype),
        scratch_types=[pltpu.VMEM((bpw,), jnp.int32),
                       pltpu.VMEM((NB, CH, D), table.dtype)]
        + [pltpu.SemaphoreType.DMA] * (2 * NB))
    def gk(table_hbm, idx_hbm, out_hbm, idx_v, bufs, *sems):
        gsem = sems[:NB]
        wsem = sems[NB:]
        wid = lax.axis_index("s") * NC + lax.axis_index("c")
        base = wid * bpw
        pltpu.sync_copy(idx_hbm.at[pl.ds(base, bpw)], idx_v)
        gds = [None] * nch
        wds = [None] * nch
        for c in range(nch):
            b = c % NB
            if c >= NB:
                wds[c - NB].wait()
            gds[c] = pltpu.async_copy(
                table_hbm.at[idx_v.at[pl.ds(c * CH, CH)]],
                bufs.at[b], gsem[b])
            p = c - AHEAD
            if p >= 0:
                gds[p].wait()
                wds[p] = pltpu.async_copy(
                    bufs.at[p % NB], out_hbm.at[pl.ds(base + p * CH, CH)],
                    wsem[p % NB])
        for p in range(max(0, nch - AHEAD), nch):
            gds[p].wait()
            wds[p] = pltpu.async_copy(
                bufs.at[p % NB], out_hbm.at[pl.ds(base + p * CH, CH)],
                wsem[p % NB])
        for p in range(max(0, nch - NB), nch):
            wds[p].wait()

    return gk(table, idx)


def _grouped_swiglu(tile_expert, x_sorted, egate_w, eup_w, edown_w):
    NPAD, D = x_sorted.shape
    E, F, _ = egate_w.shape
    nt = NPAD // _TILE

    def body(te_ref, xs_ref, gw_ref, uw_ref, dw_ref, o_ref):
        xb = xs_ref[...].astype(jnp.bfloat16)
        g = lax.dot_general(xb, gw_ref[0].astype(jnp.bfloat16),
                            (((1,), (1,)), ((), ())),
                            preferred_element_type=jnp.float32)
        u = lax.dot_general(xb, uw_ref[0].astype(jnp.bfloat16),
                            (((1,), (1,)), ((), ())),
                            preferred_element_type=jnp.float32)
        h = (g * jax.nn.sigmoid(g) * u).astype(jnp.bfloat16)
        o_ref[...] = lax.dot_general(h, dw_ref[0].astype(jnp.bfloat16),
                                     (((1,), (1,)), ((), ())),
                                     preferred_element_type=jnp.float32)

    grid_spec = pltpu.PrefetchScalarGridSpec(
        num_scalar_prefetch=1,
        grid=(nt,),
        in_specs=[pl.BlockSpec((_TILE, D), lambda i, te: (i, 0)),
                  pl.BlockSpec((1, F, D), lambda i, te: (te[i], 0, 0)),
                  pl.BlockSpec((1, F, D), lambda i, te: (te[i], 0, 0)),
                  pl.BlockSpec((1, D, F), lambda i, te: (te[i], 0, 0))],
        out_specs=pl.BlockSpec((_TILE, D), lambda i, te: (i, 0)),
    )
    return pl.pallas_call(
        body, grid_spec=grid_spec,
        out_shape=jax.ShapeDtypeStruct((NPAD, D), jnp.float32),
    )(tile_expert, x_sorted, egate_w, eup_w, edown_w)


def _shared_combine(x2d, gw_bf, uw_bf, dw_bf, alpha, routed):
    N, D = x2d.shape
    F = gw_bf.shape[0]
    TB = 256

    def body(x_ref, gw_ref, uw_ref, dw_ref, a_ref, r_ref, y_ref):
        xb = x_ref[...].astype(jnp.bfloat16)
        g = lax.dot_general(xb, gw_ref[...], (((1,), (1,)), ((), ())),
                            preferred_element_type=jnp.float32)
        u = lax.dot_general(xb, uw_ref[...], (((1,), (1,)), ((), ())),
                            preferred_element_type=jnp.float32)
        h = (g * jax.nn.sigmoid(g) * u).astype(jnp.bfloat16)
        sh = lax.dot_general(h, dw_ref[...], (((1,), (1,)), ((), ())),
                             preferred_element_type=jnp.float32)
        a = a_ref[...]
        y_ref[...] = a * sh + (1.0 - a) * r_ref[...].astype(jnp.float32)

    return pl.pallas_call(
        body,
        grid=(N // TB,),
        in_specs=[pl.BlockSpec((TB, D), lambda i: (i, 0)),
                  pl.BlockSpec((F, D), lambda i: (0, 0)),
                  pl.BlockSpec((F, D), lambda i: (0, 0)),
                  pl.BlockSpec((D, F), lambda i: (0, 0)),
                  pl.BlockSpec((TB, 1), lambda i: (i, 0)),
                  pl.BlockSpec((TB, D), lambda i: (i, 0))],
        out_specs=pl.BlockSpec((TB, D), lambda i: (i, 0)),
        out_shape=jax.ShapeDtypeStruct((N, D), jnp.float32),
    )(x2d, gw_bf, uw_bf, dw_bf, alpha, routed)


def kernel(x, router_w, egate_w, eup_w, edown_w,
           sh_gate_w, sh_up_w, sh_down_w, shg_w, shg_b):
    B, S, D = x.shape
    N = B * S
    E = router_w.shape[0]
    x2d = x.reshape(N, D)
    NPAD = N + E * _TILE

    e_idx, alpha = _router_alpha(x2d, router_w, shg_w, shg_b.reshape(1, 1))
    dst, tile_expert = _build_dispatch(e_idx, E, _TILE, NPAD)

    info = plsc.get_sparse_core_info()
    NW = info.num_cores * info.num_subcores
    x_sorted = _sc_scatter_rows(x2d, dst.reshape(NW, -1, 32), NPAD)
    out_pad = _grouped_swiglu(tile_expert, x_sorted, egate_w, eup_w, edown_w)
    routed = _sc_gather(out_pad, dst)

    y2d = _shared_combine(x2d,
                          sh_gate_w.astype(jnp.bfloat16),
                          sh_up_w.astype(jnp.bfloat16),
                          sh_down_w.astype(jnp.bfloat16),
                          alpha, routed)
    return y2d.reshape(B, S, D)


# speedup vs baseline: 1.6720x; 1.0401x over previous
import functools

import jax
import jax.numpy as jnp
from jax import lax
from jax.experimental import pallas as pl
from jax.experimental.pallas import tpu as pltpu
from jax.experimental.pallas import tpu_sc as plsc

_TILE = 256


def _pack_bf16(xb):
    H = xb.shape[1] // 2
    xu = lax.bitcast_convert_type(xb, jnp.uint16)
    lo = xu[:, :H].astype(jnp.uint32)
    hi = xu[:, H:].astype(jnp.uint32)
    return lax.bitcast_convert_type((hi << 16) | lo, jnp.int32)


def _unpack_bf16(xp):
    p = lax.bitcast_convert_type(xp, jnp.uint32)
    lo = lax.bitcast_convert_type((p & 0xFFFF).astype(jnp.uint16),
                                  jnp.bfloat16)
    hi = lax.bitcast_convert_type((p >> 16).astype(jnp.uint16), jnp.bfloat16)
    return jnp.concatenate([lo, hi], axis=1)


def _router_alpha(x2d, router_w, shg_w, shg_b2):
    N, D = x2d.shape
    E = router_w.shape[0]
    TB = 1024

    def body(x_ref, w_ref, sg_ref, sb_ref, eo_ref, ao_ref, xp_ref):
        xf = x_ref[...]
        xb = xf.astype(jnp.bfloat16)
        xp_ref[...] = _pack_bf16(xb)
        logits = lax.dot_general(
            xb, w_ref[...].astype(jnp.bfloat16),
            (((1,), (1,)), ((), ())),
            preferred_element_type=jnp.float32)
        maxv = jnp.max(logits, axis=1, keepdims=True)
        ids = lax.broadcasted_iota(jnp.int32, logits.shape, 1)
        eo_ref[...] = jnp.min(jnp.where(logits >= maxv, ids, E),
                              axis=1, keepdims=True)
        glogit = jnp.sum(xf * sg_ref[...], axis=1, keepdims=True)
        ao_ref[...] = jax.nn.sigmoid(glogit + sb_ref[0, 0])

    eo, ao, xp = pl.pallas_call(
        body,
        grid=(N // TB,),
        in_specs=[pl.BlockSpec((TB, D), lambda i: (i, 0)),
                  pl.BlockSpec((E, D), lambda i: (0, 0)),
                  pl.BlockSpec((1, D), lambda i: (0, 0)),
                  pl.BlockSpec((1, 1), lambda i: (0, 0))],
        out_specs=[pl.BlockSpec((TB, 1), lambda i: (i, 0)),
                   pl.BlockSpec((TB, 1), lambda i: (i, 0)),
                   pl.BlockSpec((TB, D // 2), lambda i: (i, 0))],
        out_shape=[jax.ShapeDtypeStruct((N, 1), jnp.int32),
                   jax.ShapeDtypeStruct((N, 1), jnp.float32),
                   jax.ShapeDtypeStruct((N, D // 2), jnp.int32)],
    )(x2d, router_w, shg_w, shg_b2)
    return eo[:, 0], ao, xp


def _build_dispatch(e_idx, E, T, NPAD):
    N = e_idx.shape[0]
    onehot = (e_idx[:, None] == jnp.arange(E, dtype=e_idx.dtype)[None, :]
              ).astype(jnp.int32)
    ranks = jnp.cumsum(onehot, axis=0) - 1
    rank_t = jnp.take_along_axis(ranks, e_idx[:, None], axis=1)[:, 0]
    counts = jnp.sum(onehot, axis=0)
    padded = ((counts + T - 1) // T) * T
    p_end = jnp.cumsum(padded)
    p_off = p_end - padded
    dst = (p_off[e_idx] + rank_t).astype(jnp.int32)
    nt = NPAD // T
    tile_expert = jnp.searchsorted(
        p_end, jnp.arange(nt, dtype=p_end.dtype) * T, side='right')
    tile_expert = jnp.minimum(tile_expert, E - 1).astype(jnp.int32)
    return dst, tile_expert


def _sc_scatter_rows(x2d, dst3d, NPAD):
    N, D = x2d.shape
    NW, nch, CH = dst3d.shape
    info = plsc.get_sparse_core_info()
    NC = info.num_cores
    NB = 3
    AHEAD = NB - 1
    bpw = nch * CH
    mesh = plsc.VectorSubcoreMesh(core_axis_name="c", subcore_axis_name="s")

    @functools.partial(
        pl.kernel, mesh=mesh,
        out_type=jax.ShapeDtypeStruct((NPAD, D), x2d.dtype),
        scratch_types=[pltpu.VMEM((nch, CH), jnp.int32),
                       pltpu.VMEM((NB, CH, D), x2d.dtype)]
        + [pltpu.SemaphoreType.DMA] * (2 * NB))
    def sk(x_hbm, idx_hbm, out_hbm, idx_v, bufs, *sems):
        gsem = sems[:NB]
        wsem = sems[NB:]
        wid = lax.axis_index("s") * NC + lax.axis_index("c")
        base = wid * bpw
        pltpu.sync_copy(idx_hbm.at[wid], idx_v)
        gds = [None] * nch
        wds = [None] * nch
        for c in range(nch):
            b = c % NB
            if c >= NB:
                wds[c - NB].wait()
            gds[c] = pltpu.async_copy(
                x_hbm.at[pl.ds(base + c * CH, CH)], bufs.at[b], gsem[b])
            p = c - AHEAD
            if p >= 0:
                gds[p].wait()
                wds[p] = pltpu.async_copy(
                    bufs.at[p % NB], out_hbm.at[idx_v.at[p]], wsem[p % NB])
        for p in range(max(0, nch - AHEAD), nch):
            gds[p].wait()
            wds[p] = pltpu.async_copy(
                bufs.at[p % NB], out_hbm.at[idx_v.at[p]], wsem[p % NB])
        for p in range(max(0, nch - NB), nch):
            wds[p].wait()

    return sk(x2d, dst3d)


def _sc_gather(table, idx):
    V, D = table.shape
    Bn = idx.shape[0]
    info = plsc.get_sparse_core_info()
    NC = info.num_cores
    NW = NC * info.num_subcores
    bpw = Bn // NW
    CH = 64 if D * table.dtype.itemsize <= 2048 else 32
    NB = 3
    nch = bpw // CH
    AHEAD = NB - 1
    mesh = plsc.VectorSubcoreMesh(core_axis_name="c", subcore_axis_name="s")

    @functools.partial(
        pl.kernel, mesh=mesh,
        out_type=jax.ShapeDtypeStruct((Bn, D), table.dtype),
        scratch_types=[pltpu.VMEM((bpw,), jnp.int32),
                       pltpu.VMEM((NB, CH, D), table.dtype)]
        + [pltpu.SemaphoreType.DMA] * (2 * NB))
    def gk(table_hbm, idx_hbm, out_hbm, idx_v, bufs, *sems):
        gsem = sems[:NB]
        wsem = sems[NB:]
        wid = lax.axis_index("s") * NC + lax.axis_index("c")
        base = wid * bpw
        pltpu.sync_copy(idx_hbm.at[pl.ds(base, bpw)], idx_v)
        gds = [None] * nch
        wds = [None] * nch
        for c in range(nch):
            b = c % NB
            if c >= NB:
                wds[c - NB].wait()
            gds[c] = pltpu.async_copy(
                table_hbm.at[idx_v.at[pl.ds(c * CH, CH)]],
                bufs.at[b], gsem[b])
            p = c - AHEAD
            if p >= 0:
                gds[p].wait()
                wds[p] = pltpu.async_copy(
                    bufs.at[p % NB], out_hbm.at[pl.ds(base + p * CH, CH)],
                    wsem[p % NB])
        for p in range(max(0, nch - AHEAD), nch):
            gds[p].wait()
            wds[p] = pltpu.async_copy(
                bufs.at[p % NB], out_hbm.at[pl.ds(base + p * CH, CH)],
                wsem[p % NB])
        for p in range(max(0, nch - NB), nch):
            wds[p].wait()

    return gk(table, idx)


def _grouped_swiglu(tile_expert, x_sorted_pk, egate_w, eup_w, edown_w):
    NPAD, H = x_sorted_pk.shape
    E, F, D = egate_w.shape
    nt = NPAD // _TILE

    def body(te_ref, xs_ref, gw_ref, uw_ref, dw_ref, o_ref):
        xb = _unpack_bf16(xs_ref[...])
        g = lax.dot_general(xb, gw_ref[0].astype(jnp.bfloat16),
                            (((1,), (1,)), ((), ())),
                            preferred_element_type=jnp.float32)
        u = lax.dot_general(xb, uw_ref[0].astype(jnp.bfloat16),
                            (((1,), (1,)), ((), ())),
                            preferred_element_type=jnp.float32)
        h = (g * jax.nn.sigmoid(g) * u).astype(jnp.bfloat16)
        o = lax.dot_general(h, dw_ref[0].astype(jnp.bfloat16),
                            (((1,), (1,)), ((), ())),
                            preferred_element_type=jnp.float32)
        o_ref[...] = _pack_bf16(o.astype(jnp.bfloat16))

    grid_spec = pltpu.PrefetchScalarGridSpec(
        num_scalar_prefetch=1,
        grid=(nt,),
        in_specs=[pl.BlockSpec((_TILE, H), lambda i, te: (i, 0)),
                  pl.BlockSpec((1, F, D), lambda i, te: (te[i], 0, 0)),
                  pl.BlockSpec((1, F, D), lambda i, te: (te[i], 0, 0)),
                  pl.BlockSpec((1, D, F), lambda i, te: (te[i], 0, 0))],
        out_specs=pl.BlockSpec((_TILE, H), lambda i, te: (i, 0)),
    )
    return pl.pallas_call(
        body, grid_spec=grid_spec,
        out_shape=jax.ShapeDtypeStruct((NPAD, H), jnp.int32),
    )(tile_expert, x_sorted_pk, egate_w, eup_w, edown_w)


def _shared_combine(x2d, gw_bf, uw_bf, dw_bf, alpha, routed):
    N, D = x2d.shape
    F = gw_bf.shape[0]
    TB = 256

    def body(x_ref, gw_ref, uw_ref, dw_ref, a_ref, rp_ref, y_ref):
        xb = x_ref[...].astype(jnp.bfloat16)
        g = lax.dot_general(xb, gw_ref[...], (((1,), (1,)), ((), ())),
                            preferred_element_type=jnp.float32)
        u = lax.dot_general(xb, uw_ref[...], (((1,), (1,)), ((), ())),
                            preferred_element_type=jnp.float32)
        h = (g * jax.nn.sigmoid(g) * u).astype(jnp.bfloat16)
        sh = lax.dot_general(h, dw_ref[...], (((1,), (1,)), ((), ())),
                             preferred_element_type=jnp.float32)
        a = a_ref[...]
        r = _unpack_bf16(rp_ref[...]).astype(jnp.float32)
        y_ref[...] = a * sh + (1.0 - a) * r

    return pl.pallas_call(
        body,
        grid=(N // TB,),
        in_specs=[pl.BlockSpec((TB, D), lambda i: (i, 0)),
                  pl.BlockSpec((F, D), lambda i: (0, 0)),
                  pl.BlockSpec((F, D), lambda i: (0, 0)),
                  pl.BlockSpec((D, F), lambda i: (0, 0)),
                  pl.BlockSpec((TB, 1), lambda i: (i, 0)),
                  pl.BlockSpec((TB, D // 2), lambda i: (i, 0))],
        out_specs=pl.BlockSpec((TB, D), lambda i: (i, 0)),
        out_shape=jax.ShapeDtypeStruct((N, D), jnp.float32),
    )(x2d, gw_bf, uw_bf, dw_bf, alpha, routed)


def kernel(x, router_w, egate_w, eup_w, edown_w,
           sh_gate_w, sh_up_w, sh_down_w, shg_w, shg_b):
    B, S, D = x.shape
    N = B * S
    E = router_w.shape[0]
    x2d = x.reshape(N, D)
    NPAD = N + E * _TILE

    e_idx, alpha, xp = _router_alpha(x2d, router_w, shg_w, shg_b.reshape(1, 1))
    dst, tile_expert = _build_dispatch(e_idx, E, _TILE, NPAD)

    info = plsc.get_sparse_core_info()
    NW = info.num_cores * info.num_subcores
    x_sorted = _sc_scatter_rows(xp, dst.reshape(NW, -1, 64), NPAD)
    out_pad = _grouped_swiglu(tile_expert, x_sorted, egate_w, eup_w, edown_w)
    routed = _sc_gather(out_pad, dst)

    y2d = _shared_combine(x2d,
                          sh_gate_w.astype(jnp.bfloat16),
                          sh_up_w.astype(jnp.bfloat16),
                          sh_down_w.astype(jnp.bfloat16),
                          alpha, routed)
    return y2d.reshape(B, S, D)


# speedup vs baseline: 1.6838x; 1.0070x over previous
import functools

import jax
import jax.numpy as jnp
from jax import lax
from jax.experimental import pallas as pl
from jax.experimental.pallas import tpu as pltpu
from jax.experimental.pallas import tpu_sc as plsc

_TILE = 256


def _pack_bf16(xb):
    H = xb.shape[1] // 2
    xu = lax.bitcast_convert_type(xb, jnp.uint16)
    lo = xu[:, :H].astype(jnp.uint32)
    hi = xu[:, H:].astype(jnp.uint32)
    return lax.bitcast_convert_type((hi << 16) | lo, jnp.int32)


def _unpack_bf16(xp):
    p = lax.bitcast_convert_type(xp, jnp.uint32)
    lo = lax.bitcast_convert_type((p & 0xFFFF).astype(jnp.uint16),
                                  jnp.bfloat16)
    hi = lax.bitcast_convert_type((p >> 16).astype(jnp.uint16), jnp.bfloat16)
    return jnp.concatenate([lo, hi], axis=1)


def _router_alpha(x2d, router_w, shg_w, shg_b2, sh_gate_w, sh_up_w,
                  sh_down_w):
    N, D = x2d.shape
    E = router_w.shape[0]
    F = sh_gate_w.shape[0]
    TB = 1024
    FB = F // (N // TB)

    def body(x_ref, w_ref, sg_ref, sb_ref, gw_ref, uw_ref, dw_ref,
             eo_ref, ao_ref, xp_ref, gwo_ref, uwo_ref, dwo_ref):
        xf = x_ref[...]
        xb = xf.astype(jnp.bfloat16)
        xp_ref[...] = _pack_bf16(xb)
        gwo_ref[...] = gw_ref[...].astype(jnp.bfloat16)
        uwo_ref[...] = uw_ref[...].astype(jnp.bfloat16)
        dwo_ref[...] = dw_ref[...].astype(jnp.bfloat16)
        logits = lax.dot_general(
            xb, w_ref[...].astype(jnp.bfloat16),
            (((1,), (1,)), ((), ())),
            preferred_element_type=jnp.float32)
        maxv = jnp.max(logits, axis=1, keepdims=True)
        ids = lax.broadcasted_iota(jnp.int32, logits.shape, 1)
        eo_ref[...] = jnp.min(jnp.where(logits >= maxv, ids, E),
                              axis=1, keepdims=True)
        glogit = jnp.sum(xf * sg_ref[...], axis=1, keepdims=True)
        ao_ref[...] = jax.nn.sigmoid(glogit + sb_ref[0, 0])

    eo, ao, xp, gw_bf, uw_bf, dw_bf = pl.pallas_call(
        body,
        grid=(N // TB,),
        in_specs=[pl.BlockSpec((TB, D), lambda i: (i, 0)),
                  pl.BlockSpec((E, D), lambda i: (0, 0)),
                  pl.BlockSpec((1, D), lambda i: (0, 0)),
                  pl.BlockSpec((1, 1), lambda i: (0, 0)),
                  pl.BlockSpec((FB, D), lambda i: (i, 0)),
                  pl.BlockSpec((FB, D), lambda i: (i, 0)),
                  pl.BlockSpec((D, FB), lambda i: (0, i))],
        out_specs=[pl.BlockSpec((TB, 1), lambda i: (i, 0)),
                   pl.BlockSpec((TB, 1), lambda i: (i, 0)),
                   pl.BlockSpec((TB, D // 2), lambda i: (i, 0)),
                   pl.BlockSpec((FB, D), lambda i: (i, 0)),
                   pl.BlockSpec((FB, D), lambda i: (i, 0)),
                   pl.BlockSpec((D, FB), lambda i: (0, i))],
        out_shape=[jax.ShapeDtypeStruct((N, 1), jnp.int32),
                   jax.ShapeDtypeStruct((N, 1), jnp.float32),
                   jax.ShapeDtypeStruct((N, D // 2), jnp.int32),
                   jax.ShapeDtypeStruct((F, D), jnp.bfloat16),
                   jax.ShapeDtypeStruct((F, D), jnp.bfloat16),
                   jax.ShapeDtypeStruct((D, F), jnp.bfloat16)],
    )(x2d, router_w, shg_w, shg_b2, sh_gate_w, sh_up_w, sh_down_w)
    return eo[:, 0], ao, xp, gw_bf, uw_bf, dw_bf


def _build_dispatch(e_idx, E, T, NPAD):
    N = e_idx.shape[0]
    onehot = (e_idx[:, None] == jnp.arange(E, dtype=e_idx.dtype)[None, :]
              ).astype(jnp.int32)
    ranks = jnp.cumsum(onehot, axis=0) - 1
    rank_t = jnp.take_along_axis(ranks, e_idx[:, None], axis=1)[:, 0]
    counts = jnp.sum(onehot, axis=0)
    padded = ((counts + T - 1) // T) * T
    p_end = jnp.cumsum(padded)
    p_off = p_end - padded
    dst = (p_off[e_idx] + rank_t).astype(jnp.int32)
    nt = NPAD // T
    tile_expert = jnp.searchsorted(
        p_end, jnp.arange(nt, dtype=p_end.dtype) * T, side='right')
    tile_expert = jnp.minimum(tile_expert, E - 1).astype(jnp.int32)
    return dst, tile_expert


def _sc_scatter_rows(x2d, dst3d, NPAD):
    N, D = x2d.shape
    NW, nch, CH = dst3d.shape
    info = plsc.get_sparse_core_info()
    NC = info.num_cores
    NB = 3
    AHEAD = NB - 1
    bpw = nch * CH
    mesh = plsc.VectorSubcoreMesh(core_axis_name="c", subcore_axis_name="s")

    @functools.partial(
        pl.kernel, mesh=mesh,
        out_type=jax.ShapeDtypeStruct((NPAD, D), x2d.dtype),
        scratch_types=[pltpu.VMEM((nch, CH), jnp.int32),
                       pltpu.VMEM((NB, CH, D), x2d.dtype)]
        + [pltpu.SemaphoreType.DMA] * (2 * NB))
    def sk(x_hbm, idx_hbm, out_hbm, idx_v, bufs, *sems):
        gsem = sems[:NB]
        wsem = sems[NB:]
        wid = lax.axis_index("s") * NC + lax.axis_index("c")
        base = wid * bpw
        pltpu.sync_copy(idx_hbm.at[wid], idx_v)
        gds = [None] * nch
        wds = [None] * nch
        for c in range(nch):
            b = c % NB
            if c >= NB:
                wds[c - NB].wait()
            gds[c] = pltpu.async_copy(
                x_hbm.at[pl.ds(base + c * CH, CH)], bufs.at[b], gsem[b])
            p = c - AHEAD
            if p >= 0:
                gds[p].wait()
                wds[p] = pltpu.async_copy(
                    bufs.at[p % NB], out_hbm.at[idx_v.at[p]], wsem[p % NB])
        for p in range(max(0, nch - AHEAD), nch):
            gds[p].wait()
            wds[p] = pltpu.async_copy(
                bufs.at[p % NB], out_hbm.at[idx_v.at[p]], wsem[p % NB])
        for p in range(max(0, nch - NB), nch):
            wds[p].wait()

    return sk(x2d, dst3d)


def _sc_gather(table, idx):
    V, D = table.shape
    Bn = idx.shape[0]
    info = plsc.get_sparse_core_info()
    NC = info.num_cores
    NW = NC * info.num_subcores
    bpw = Bn // NW
    CH = 64 if D * table.dtype.itemsize <= 2048 else 32
    NB = 3
    nch = bpw // CH
    AHEAD = NB - 1
    mesh = plsc.VectorSubcoreMesh(core_axis_name="c", subcore_axis_name="s")

    @functools.partial(
        pl.kernel, mesh=mesh,
        out_type=jax.ShapeDtypeStruct((Bn, D), table.dtype),
        scratch_types=[pltpu.VMEM((bpw,), jnp.int32),
                       pltpu.VMEM((NB, CH, D), table.dtype)]
        + [pltpu.SemaphoreType.DMA] * (2 * NB))
    def gk(table_hbm, idx_hbm, out_hbm, idx_v, bufs, *sems):
        gsem = sems[:NB]
        wsem = sems[NB:]
        wid = lax.axis_index("s") * NC + lax.axis_index("c")
        base = wid * bpw
        pltpu.sync_copy(idx_hbm.at[pl.ds(base, bpw)], idx_v)
        gds = [None] * nch
        wds = [None] * nch
        for c in range(nch):
            b = c % NB
            if c >= NB:
                wds[c - NB].wait()
            gds[c] = pltpu.async_copy(
                table_hbm.at[idx_v.at[pl.ds(c * CH, CH)]],
                bufs.at[b], gsem[b])
            p = c - AHEAD
            if p >= 0:
                gds[p].wait()
                wds[p] = pltpu.async_copy(
                    bufs.at[p % NB], out_hbm.at[pl.ds(base + p * CH, CH)],
                    wsem[p % NB])
        for p in range(max(0, nch - AHEAD), nch):
            gds[p].wait()
            wds[p] = pltpu.async_copy(
                bufs.at[p % NB], out_hbm.at[pl.ds(base + p * CH, CH)],
                wsem[p % NB])
        for p in range(max(0, nch - NB), nch):
            wds[p].wait()

    return gk(table, idx)


def _grouped_swiglu(tile_expert, x_sorted_pk, egate_w, eup_w, edown_w):
    NPAD, H = x_sorted_pk.shape
    E, F, D = egate_w.shape
    nt = NPAD // _TILE

    def body(te_ref, xs_ref, gw_ref, uw_ref, dw_ref, o_ref):
        xb = _unpack_bf16(xs_ref[...])
        g = lax.dot_general(xb, gw_ref[0].astype(jnp.bfloat16),
                            (((1,), (1,)), ((), ())),
                            preferred_element_type=jnp.float32)
        u = lax.dot_general(xb, uw_ref[0].astype(jnp.bfloat16),
                            (((1,), (1,)), ((), ())),
                            preferred_element_type=jnp.float32)
        h = (g * jax.nn.sigmoid(g) * u).astype(jnp.bfloat16)
        o = lax.dot_general(h, dw_ref[0].astype(jnp.bfloat16),
                            (((1,), (1,)), ((), ())),
                            preferred_element_type=jnp.float32)
        o_ref[...] = _pack_bf16(o.astype(jnp.bfloat16))

    grid_spec = pltpu.PrefetchScalarGridSpec(
        num_scalar_prefetch=1,
        grid=(nt,),
        in_specs=[pl.BlockSpec((_TILE, H), lambda i, te: (i, 0)),
                  pl.BlockSpec((1, F, D), lambda i, te: (te[i], 0, 0)),
                  pl.BlockSpec((1, F, D), lambda i, te: (te[i], 0, 0)),
                  pl.BlockSpec((1, D, F), lambda i, te: (te[i], 0, 0))],
        out_specs=pl.BlockSpec((_TILE, H), lambda i, te: (i, 0)),
    )
    return pl.pallas_call(
        body, grid_spec=grid_spec,
        out_shape=jax.ShapeDtypeStruct((NPAD, H), jnp.int32),
    )(tile_expert, x_sorted_pk, egate_w, eup_w, edown_w)


def _shared_combine(x2d, gw_bf, uw_bf, dw_bf, alpha, routed):
    N, D = x2d.shape
    F = gw_bf.shape[0]
    TB = 256

    def body(x_ref, gw_ref, uw_ref, dw_ref, a_ref, rp_ref, y_ref):
        xb = x_ref[...].astype(jnp.bfloat16)
        g = lax.dot_general(xb, gw_ref[...], (((1,), (1,)), ((), ())),
                            preferred_element_type=jnp.float32)
        u = lax.dot_general(xb, uw_ref[...], (((1,), (1,)), ((), ())),
                            preferred_element_type=jnp.float32)
        h = (g * jax.nn.sigmoid(g) * u).astype(jnp.bfloat16)
        sh = lax.dot_general(h, dw_ref[...], (((1,), (1,)), ((), ())),
                             preferred_element_type=jnp.float32)
        a = a_ref[...]
        r = _unpack_bf16(rp_ref[...]).astype(jnp.float32)
        y_ref[...] = a * sh + (1.0 - a) * r

    return pl.pallas_call(
        body,
        grid=(N // TB,),
        in_specs=[pl.BlockSpec((TB, D), lambda i: (i, 0)),
                  pl.BlockSpec((F, D), lambda i: (0, 0)),
                  pl.BlockSpec((F, D), lambda i: (0, 0)),
                  pl.BlockSpec((D, F), lambda i: (0, 0)),
                  pl.BlockSpec((TB, 1), lambda i: (i, 0)),
                  pl.BlockSpec((TB, D // 2), lambda i: (i, 0))],
        out_specs=pl.BlockSpec((TB, D), lambda i: (i, 0)),
        out_shape=jax.ShapeDtypeStruct((N, D), jnp.float32),
    )(x2d, gw_bf, uw_bf, dw_bf, alpha, routed)


def kernel(x, router_w, egate_w, eup_w, edown_w,
           sh_gate_w, sh_up_w, sh_down_w, shg_w, shg_b):
    B, S, D = x.shape
    N = B * S
    E = router_w.shape[0]
    x2d = x.reshape(N, D)
    NPAD = N + E * _TILE

    e_idx, alpha, xp, gw_bf, uw_bf, dw_bf = _router_alpha(
        x2d, router_w, shg_w, shg_b.reshape(1, 1),
        sh_gate_w, sh_up_w, sh_down_w)
    dst, tile_expert = _build_dispatch(e_idx, E, _TILE, NPAD)

    info = plsc.get_sparse_core_info()
    NW = info.num_cores * info.num_subcores
    x_sorted = _sc_scatter_rows(xp, dst.reshape(NW, -1, 64), NPAD)
    out_pad = _grouped_swiglu(tile_expert, x_sorted, egate_w, eup_w, edown_w)
    routed = _sc_gather(out_pad, dst)

    y2d = _shared_combine(x2d, gw_bf, uw_bf, dw_bf, alpha, routed)
    return y2d.reshape(B, S, D)
